# baseline (speedup 1.0000x reference)
import functools

import jax
import jax.numpy as jnp
from jax.experimental import pallas as pl

_T = 0.05
_BN = 2048


def _matmul_kernel(x_ref, mem_ref, out_ref):
    acc = jax.lax.dot_general(
        x_ref[...],
        mem_ref[...],
        dimension_numbers=(((1,), (1,)), ((), ())),
        preferred_element_type=jnp.float32,
    )
    out_ref[...] = acc / _T


@jax.jit
def kernel(x, memory):
    m, k = x.shape
    n = memory.shape[0]
    grid = (pl.cdiv(n, _BN),)
    return pl.pallas_call(
        _matmul_kernel,
        grid=grid,
        in_specs=[
            pl.BlockSpec((m, k), lambda i: (0, 0)),
            pl.BlockSpec((_BN, k), lambda i: (i, 0)),
        ],
        out_specs=pl.BlockSpec((m, _BN), lambda i: (0, i)),
        out_shape=jax.ShapeDtypeStruct((m, n), jnp.float32),
    )(x, memory)


# speedup vs baseline: 1.0041x; 1.0041x over previous
import functools

import jax
import jax.numpy as jnp
from jax.experimental import pallas as pl

_T = 0.05
_BN = 4096


def _matmul_kernel(x_ref, mem_ref, out_ref):
    acc = jax.lax.dot_general(
        x_ref[...],
        mem_ref[...],
        dimension_numbers=(((1,), (1,)), ((), ())),
        preferred_element_type=jnp.float32,
    )
    out_ref[...] = acc / _T


@jax.jit
def kernel(x, memory):
    m, k = x.shape
    n = memory.shape[0]
    grid = (pl.cdiv(n, _BN),)
    return pl.pallas_call(
        _matmul_kernel,
        grid=grid,
        in_specs=[
            pl.BlockSpec((m, k), lambda i: (0, 0)),
            pl.BlockSpec((_BN, k), lambda i: (i, 0)),
        ],
        out_specs=pl.BlockSpec((m, _BN), lambda i: (0, i)),
        out_shape=jax.ShapeDtypeStruct((m, n), jnp.float32),
    )(x, memory)


# speedup vs baseline: 1.0063x; 1.0022x over previous
import functools

import jax
import jax.numpy as jnp
from jax.experimental import pallas as pl
from jax.experimental.pallas import tpu as pltpu

_T = 0.05
_BN = 4096


def _matmul_kernel(x_ref, mem_ref, out_ref):
    acc = jax.lax.dot_general(
        x_ref[...],
        mem_ref[...],
        dimension_numbers=(((1,), (1,)), ((), ())),
        preferred_element_type=jnp.float32,
    )
    out_ref[...] = acc / _T


@jax.jit
def kernel(x, memory):
    m, k = x.shape
    n = memory.shape[0]
    grid = (pl.cdiv(n, _BN),)
    return pl.pallas_call(
        _matmul_kernel,
        grid=grid,
        in_specs=[
            pl.BlockSpec((m, k), lambda i: (0, 0)),
            pl.BlockSpec((_BN, k), lambda i: (i, 0)),
        ],
        out_specs=pl.BlockSpec((m, _BN), lambda i: (0, i)),
        out_shape=jax.ShapeDtypeStruct((m, n), jnp.float32),
        compiler_params=pltpu.CompilerParams(
            dimension_semantics=("parallel",),
        ),
    )(x, memory)


# speedup vs baseline: 1.0861x; 1.0793x over previous
import jax
import jax.numpy as jnp
from jax.experimental import pallas as pl
from jax.experimental.pallas import tpu as pltpu

_T = 0.05
_BM = 64


def _matmul_kernel(x_ref, memt_ref, out_ref):
    acc = jax.lax.dot_general(
        x_ref[...],
        memt_ref[...],
        dimension_numbers=(((1,), (0,)), ((), ())),
        preferred_element_type=jnp.float32,
    )
    out_ref[...] = acc / _T


@jax.jit
def kernel(x, memory):
    m, k = x.shape
    n = memory.shape[0]
    memt = memory.T
    grid = (m // _BM,)
    return pl.pallas_call(
        _matmul_kernel,
        grid=grid,
        in_specs=[
            pl.BlockSpec((_BM, k), lambda i: (i, 0)),
            pl.BlockSpec((k, n), lambda i: (0, 0)),
        ],
        out_specs=pl.BlockSpec((_BM, n), lambda i: (i, 0)),
        out_shape=jax.ShapeDtypeStruct((m, n), jnp.float32),
        compiler_params=pltpu.CompilerParams(
            dimension_semantics=("parallel",),
            vmem_limit_bytes=63 * 1024 * 1024,
        ),
    )(x, memt)


# speedup vs baseline: 1.0999x; 1.0126x over previous
import jax
import jax.numpy as jnp
from jax.experimental import pallas as pl
from jax.experimental.pallas import tpu as pltpu

_T = 0.05
_BM = 16
_NBUF = 6


def _matmul_kernel(x_ref, memt_ref, out_hbm, scratch, sems):
    i = pl.program_id(0)
    g = pl.num_programs(0)
    slot = jax.lax.rem(i, _NBUF)

    @pl.when(i >= _NBUF)
    def _():
        prev = i - _NBUF
        pltpu.make_async_copy(
            scratch.at[jax.lax.rem(prev, _NBUF)],
            out_hbm.at[pl.ds(prev * _BM, _BM), :],
            sems.at[jax.lax.rem(prev, _NBUF)],
        ).wait()

    acc = jax.lax.dot_general(
        x_ref[...],
        memt_ref[...],
        dimension_numbers=(((1,), (0,)), ((), ())),
        preferred_element_type=jnp.float32,
    )
    scratch[slot] = acc / _T

    pltpu.make_async_copy(
        scratch.at[slot],
        out_hbm.at[pl.ds(i * _BM, _BM), :],
        sems.at[slot],
    ).start()

    @pl.when(i == g - 1)
    def _():
        for j in range(_NBUF):
            s = i - (_NBUF - 1) + j
            @pl.when(s >= 0)
            def _():
                pltpu.make_async_copy(
                    scratch.at[jax.lax.rem(s, _NBUF)],
                    out_hbm.at[pl.ds(s * _BM, _BM), :],
                    sems.at[jax.lax.rem(s, _NBUF)],
                ).wait()


@jax.jit
def kernel(x, memory):
    m, k = x.shape
    n = memory.shape[0]
    memt = memory.T
    grid = (m // _BM,)
    return pl.pallas_call(
        _matmul_kernel,
        grid=grid,
        in_specs=[
            pl.BlockSpec((_BM, k), lambda i: (i, 0)),
            pl.BlockSpec((k, n), lambda i: (0, 0)),
        ],
        out_specs=pl.BlockSpec(memory_space=pltpu.MemorySpace.HBM),
        out_shape=jax.ShapeDtypeStruct((m, n), jnp.float32),
        scratch_shapes=[
            pltpu.VMEM((_NBUF, _BM, n), jnp.float32),
            pltpu.SemaphoreType.DMA((_NBUF,)),
        ],
        compiler_params=pltpu.CompilerParams(
            dimension_semantics=("arbitrary",),
            vmem_limit_bytes=63 * 1024 * 1024,
        ),
    )(x, memt)
